# baseline (speedup 1.0000x reference)
import functools

import jax
import jax.numpy as jnp
from jax import lax
from jax.experimental import pallas as pl
from jax.experimental.pallas import tpu as pltpu
from jax.experimental.pallas import tpu_sc as plsc

EMBED_DIM = 64
SPLIT = (104, 96)
NLANE = 16


@functools.lru_cache(maxsize=None)
def _build(batch, hist):
    info = plsc.get_sparse_core_info()
    nc, ns = info.num_cores, info.num_subcores
    nw = nc * ns
    rows_per_w = batch // nw
    assert rows_per_w * nw == batch and sum(SPLIT) == hist
    per_w = rows_per_w * hist
    vpr = EMBED_DIM // NLANE
    wrow = hist // 2

    mesh = plsc.VectorSubcoreMesh(core_axis_name="c", subcore_axis_name="s")

    @functools.partial(
        pl.kernel,
        out_type=jax.ShapeDtypeStruct((batch * wrow, 2 * EMBED_DIM),
                                      jnp.float32),
        mesh=mesh,
        scratch_types=[
            pltpu.VMEM((per_w,), jnp.int32),
            [pltpu.VMEM((hist, 2 * EMBED_DIM), jnp.float32)
             for _ in range(2)],
            [pltpu.VMEM((wrow, 2 * EMBED_DIM), jnp.float32)
             for _ in range(2)],
            [pltpu.VMEM((hist,), jnp.int32) for _ in range(2)],
            [pltpu.SemaphoreType.DMA for _ in range(2)],
            [pltpu.SemaphoreType.DMA for _ in range(2)],
            [pltpu.SemaphoreType.DMA for _ in range(2)],
        ],
        compiler_params=pltpu.CompilerParams(use_tc_tiling_on_sc=False),
    )
    def gather_kernel(idxh_hbm, off_hbm, t2_hbm, out_hbm,
                      idx_v, pairs, comp, offv, gsem, wsem, osem):
        wid = lax.axis_index("s") * nc + lax.axis_index("c")
        base = wid * rows_per_w

        pltpu.sync_copy(idxh_hbm.at[pl.ds(base * hist, per_w)], idx_v)

        def fire(r, b):
            pltpu.async_copy(off_hbm.at[pl.ds((base + r) * hist, hist)],
                             offv[b], osem[b])
            off = 0
            for n in SPLIT:
                pltpu.async_copy(
                    t2_hbm.at[idx_v.at[pl.ds(r * hist + off, n)]],
                    pairs[b].at[pl.ds(off, n)],
                    gsem[b])
                off += n

        def drain_gathers(r, b):
            pltpu.make_async_copy(off_hbm.at[pl.ds((base + r) * hist, hist)],
                                  offv[b], osem[b]).wait()
            off = 0
            for n in SPLIT:
                pltpu.make_async_copy(
                    t2_hbm.at[idx_v.at[pl.ds(r * hist + off, n)]],
                    pairs[b].at[pl.ds(off, n)],
                    gsem[b]).wait()
                off += n

        def compact(b):
            src = pairs[b]
            dst = comp[b]

            def group(g0):
                vo = offv[b][pl.ds(g0, NLANE)]
                for l in range(NLANE):
                    i = g0 + l
                    o = vo[l]
                    i2 = lax.div(i, 2)
                    c0 = lax.rem(i, 2) * EMBED_DIM
                    for j in range(vpr):
                        dst[i2, pl.ds(c0 + j * NLANE, NLANE)] = (
                            src[i, pl.ds(o + j * NLANE, NLANE)])

            @pl.loop(0, hist - NLANE, step=NLANE)
            def _(g0):
                group(g0)

            group(hist - NLANE)

        def start_write(r, b):
            pltpu.async_copy(comp[b],
                             out_hbm.at[pl.ds((base + r) * wrow, wrow)],
                             wsem[b])

        def wait_write(r, b):
            pltpu.make_async_copy(comp[b],
                                  out_hbm.at[pl.ds((base + r) * wrow, wrow)],
                                  wsem[b]).wait()

        fire(0, 0)

        @pl.loop(0, rows_per_w, step=2)
        def _(r0):
            for b in range(2):
                r = r0 + b
                nb = 1 - b

                @pl.when(r + 1 < rows_per_w)
                def _():
                    fire(r + 1, nb)

                drain_gathers(r, b)

                @pl.when(r >= 2)
                def _():
                    wait_write(r - 2, b)

                compact(b)
                start_write(r, b)

        for b in range(2):
            wait_write(rows_per_w - 2 + b, b)

    return gather_kernel


def kernel(indices, embeddings):
    batch, hist = indices.shape
    run = _build(batch, hist)
    idx_flat = indices.reshape(-1)
    idxh = idx_flat >> 1
    off = (idx_flat & 1) << 6
    t2 = embeddings.reshape(embeddings.shape[0] // 2, 2 * EMBED_DIM)
    out2 = run(idxh, off, t2)
    return out2.reshape(batch, hist, EMBED_DIM)


# speedup vs baseline: 1.6070x; 1.6070x over previous
import functools

import jax
import jax.numpy as jnp
from jax import lax
from jax.experimental import pallas as pl
from jax.experimental.pallas import tpu as pltpu
from jax.experimental.pallas import tpu_sc as plsc

EMBED_DIM = 64
OUT_W = 2 * EMBED_DIM
CHUNK = 128
K = 4
NBUF = 2


@functools.lru_cache(maxsize=None)
def _build(n_total):
    info = plsc.get_sparse_core_info()
    nc, ns = info.num_cores, info.num_subcores
    nw = nc * ns
    per_w = n_total // nw
    group = K * CHUNK
    assert per_w * nw == n_total and per_w % group == 0
    n_chunks = per_w // CHUNK
    n_groups = per_w // group
    assert n_groups % NBUF == 0

    mesh = plsc.VectorSubcoreMesh(core_axis_name="c", subcore_axis_name="s")

    @functools.partial(
        pl.kernel,
        out_type=jax.ShapeDtypeStruct((n_total, OUT_W), jnp.float32),
        mesh=mesh,
        scratch_types=[
            pltpu.VMEM((n_chunks, CHUNK), jnp.int32),
            [pltpu.VMEM((group, EMBED_DIM), jnp.float32) for _ in range(NBUF)],
            [pltpu.SemaphoreType.DMA for _ in range(NBUF)],
            [pltpu.SemaphoreType.DMA for _ in range(NBUF)],
        ],
        compiler_params=pltpu.CompilerParams(use_tc_tiling_on_sc=False),
    )
    def gather_kernel(idx_hbm, table_hbm, out_hbm, idx_v, rows, gsem, wsem):
        wid = lax.axis_index("s") * nc + lax.axis_index("c")
        base = wid * per_w

        pltpu.sync_copy(idx_hbm.at[wid], idx_v)

        def out_slice(g):
            return out_hbm.at[pl.ds(base + g * group, group),
                              pl.ds(0, EMBED_DIM)]

        def fire(g, b):
            for t in range(K):
                pltpu.async_copy(
                    table_hbm.at[idx_v.at[g * K + t]],
                    rows[b].at[pl.ds(t * CHUNK, CHUNK)],
                    gsem[b])

        def drain_gathers(g, b):
            for t in range(K):
                pltpu.make_async_copy(
                    table_hbm.at[idx_v.at[g * K + t]],
                    rows[b].at[pl.ds(t * CHUNK, CHUNK)],
                    gsem[b]).wait()

        def start_write(g, b):
            pltpu.async_copy(rows[b], out_slice(g), wsem[b])

        def wait_write(g, b):
            pltpu.make_async_copy(rows[b], out_slice(g), wsem[b]).wait()

        fire(0, 0)

        @pl.loop(0, n_groups, step=NBUF)
        def _(g0):
            for b in range(NBUF):
                g = g0 + b
                drain_gathers(g, b)
                nb = (b + 1) % NBUF

                @pl.when(g + 1 < n_groups)
                def _():
                    @pl.when(g + 1 >= NBUF)
                    def _():
                        wait_write(g + 1 - NBUF, nb)
                    fire(g + 1, nb)

                start_write(g, b)

        for b in range(NBUF):
            wait_write(n_groups - NBUF + b, b)

    return gather_kernel, nw, n_chunks


def kernel(indices, embeddings):
    batch, hist = indices.shape
    n_total = batch * hist
    run, nw, n_chunks = _build(n_total)
    idx3 = indices.reshape(nw, n_chunks, CHUNK)
    out2 = run(idx3, embeddings)
    return out2[:, :EMBED_DIM].reshape(batch, hist, EMBED_DIM)
